# baseline (device time: 38669 ns/iter reference)
import jax
import jax.numpy as jnp
from jax import lax
from jax.experimental import pallas as pl
from jax.experimental.pallas import tpu as pltpu

N_DEV = 4
B, SQ, SKV, DH = 2, 512, 512, 64
HQ_LOCAL = 8
HD_LOCAL = HQ_LOCAL * DH
D_MODEL = 768
BLK = 64
HALF = SQ // 2
QTR = SQ // 4


def kernel(x, Wq, K_ext, V_ext, Wo):
    my = lax.axis_index("i")
    Wq_my = lax.dynamic_slice_in_dim(Wq, my * HD_LOCAL, HD_LOCAL, axis=1) * 0.125
    Wo_my = lax.dynamic_slice_in_dim(Wo, my * HD_LOCAL, HD_LOCAL, axis=0)
    x_b = x.astype(jnp.bfloat16)
    Wq_my = Wq_my.astype(jnp.bfloat16)
    Wo_my = Wo_my.astype(jnp.bfloat16)
    K_t = K_ext.transpose(0, 2, 1, 3).astype(jnp.bfloat16)
    V_t = V_ext.transpose(0, 2, 1, 3).astype(jnp.bfloat16)

    def body(x_ref, wq_ref, k_ref, v_ref, wo_ref, out_ref,
             q_ref, ctx_ref, mask_ref, acc_ref, rs1a, rs2a, rs1b, rs2b,
             send_sems, recv_sems):
        my_pos = lax.axis_index("i")
        yc = ((my_pos + 1) // 2) % 2
        xc = my_pos // 2
        p_y = my_pos ^ 1
        p_x = 3 - my_pos

        barrier_sem = pltpu.get_barrier_semaphore()
        for nbr in (p_y, p_x):
            pl.semaphore_signal(
                barrier_sem, inc=1,
                device_id=(nbr,), device_id_type=pl.DeviceIdType.MESH,
            )
        pl.semaphore_wait(barrier_sem, 2)

        qb = lax.broadcasted_iota(jnp.int32, (SQ, SKV), 0) // BLK
        kb = lax.broadcasted_iota(jnp.int32, (SQ, SKV), 1) // BLK
        keep = (qb == kb) | (kb == 0) | ((qb + kb) % 3 == 0)
        mask_ref[...] = keep.astype(jnp.float32)

        def qproj(b):
            q_ref[...] = jnp.dot(
                x_ref[b], wq_ref[...], preferred_element_type=jnp.float32
            ).astype(jnp.bfloat16)

        def chunk(b, off, rows):
            for h in range(HQ_LOCAL):
                q = q_ref[pl.ds(off, rows), h * DH:(h + 1) * DH]
                s = lax.dot_general(
                    q, k_ref[b, h], (((1,), (1,)), ((), ())),
                    preferred_element_type=jnp.float32,
                )
                w = jnp.exp(s) * mask_ref[pl.ds(off, rows), :]
                denom = jnp.sum(w, axis=-1, keepdims=True)
                ctx = jnp.dot(
                    w.astype(jnp.bfloat16), v_ref[b, h],
                    preferred_element_type=jnp.float32,
                ) / denom
                ctx_ref[pl.ds(off, rows), h * DH:(h + 1) * DH] = (
                    ctx.astype(jnp.bfloat16))
            acc_ref[b, pl.ds(off, rows), :] = jnp.dot(
                ctx_ref[pl.ds(off, rows), :], wo_ref[...],
                preferred_element_type=jnp.float32,
            ).astype(jnp.bfloat16)

        streams = (
            (0, yc, xc, (p_y, p_x, p_x, p_y), rs1a, rs2a),
            (1, xc, yc, (p_x, p_y, p_y, p_x), rs1b, rs2b),
        )

        def make_round(s, k):
            b, h, q, parts, rs1, rs2 = streams[s]
            half0 = h * HALF
            own = half0 + q * QTR
            if k == 0:
                src = acc_ref.at[b, pl.ds((1 - h) * HALF, HALF), :]
                dst = rs1.at[...]
            elif k == 1:
                src = acc_ref.at[b, pl.ds(half0 + (1 - q) * QTR, QTR), :]
                dst = rs2.at[...]
            elif k == 2:
                src = acc_ref.at[b, pl.ds(own, QTR), :]
                dst = acc_ref.at[b, pl.ds(own, QTR), :]
            else:
                src = acc_ref.at[b, pl.ds(half0, HALF), :]
                dst = acc_ref.at[b, pl.ds(half0, HALF), :]
            return pltpu.make_async_remote_copy(
                src_ref=src, dst_ref=dst,
                send_sem=send_sems.at[s, k], recv_sem=recv_sems.at[s, k],
                device_id=(parts[k],), device_id_type=pl.DeviceIdType.MESH,
            )

        def apply_round(s, k):
            b, h, q, parts, rs1, rs2 = streams[s]
            half0 = h * HALF
            if k == 0:
                acc_ref[b, pl.ds(half0, HALF), :] += rs1[...]
            elif k == 1:
                own = half0 + q * QTR
                acc_ref[b, pl.ds(own, QTR), :] += rs2[...]

        h_a = yc
        h_b = xc

        qproj(0)
        chunk(0, (1 - h_a) * HALF, HALF)
        ra1 = make_round(0, 0)
        ra1.start()
        chunk(0, h_a * HALF, HALF)
        qproj(1)
        chunk(1, (1 - h_b) * HALF, HALF)
        rb1 = make_round(1, 0)
        rb1.start()
        ra1.wait()
        apply_round(0, 0)
        ra2 = make_round(0, 1)
        ra2.start()
        chunk(1, h_b * HALF, HALF)
        rb1.wait()
        apply_round(1, 0)
        rb2 = make_round(1, 1)
        rb2.start()
        ra2.wait()
        apply_round(0, 1)
        ra3 = make_round(0, 2)
        ra3.start()
        rb2.wait()
        apply_round(1, 1)
        rb3 = make_round(1, 2)
        rb3.start()
        ra3.wait()
        ra4 = make_round(0, 3)
        ra4.start()
        rb3.wait()
        rb4 = make_round(1, 3)
        rb4.start()
        ra4.wait()
        out_ref[0] = acc_ref[0].astype(jnp.float32)
        rb4.wait()
        out_ref[1] = acc_ref[1].astype(jnp.float32)

    return pl.pallas_call(
        body,
        out_shape=jax.ShapeDtypeStruct((B, SQ, D_MODEL), jnp.float32),
        in_specs=[pl.BlockSpec(memory_space=pltpu.VMEM)] * 5,
        out_specs=pl.BlockSpec(memory_space=pltpu.VMEM),
        scratch_shapes=[
            pltpu.VMEM((SQ, HD_LOCAL), jnp.bfloat16),
            pltpu.VMEM((SQ, HD_LOCAL), jnp.bfloat16),
            pltpu.VMEM((SQ, SKV), jnp.float32),
            pltpu.VMEM((B, SQ, D_MODEL), jnp.bfloat16),
            pltpu.VMEM((HALF, D_MODEL), jnp.bfloat16),
            pltpu.VMEM((QTR, D_MODEL), jnp.bfloat16),
            pltpu.VMEM((HALF, D_MODEL), jnp.bfloat16),
            pltpu.VMEM((QTR, D_MODEL), jnp.bfloat16),
            pltpu.SemaphoreType.DMA((2, 4)),
            pltpu.SemaphoreType.DMA((2, 4)),
        ],
        compiler_params=pltpu.CompilerParams(collective_id=0),
    )(x_b, Wq_my, K_t, V_t, Wo_my)


# device time: 35943 ns/iter; 1.0758x vs baseline; 1.0758x over previous
import jax
import jax.numpy as jnp
from jax import lax
from jax.experimental import pallas as pl
from jax.experimental.pallas import tpu as pltpu

N_DEV = 4
B, SQ, SKV, DH = 2, 512, 512, 64
HQ_LOCAL = 8
HD_LOCAL = HQ_LOCAL * DH
D_MODEL = 768
BLK = 64
HALF = SQ // 2
QTR = SQ // 4


def kernel(x, Wq, K_ext, V_ext, Wo):
    my = lax.axis_index("i")
    Wq_my = lax.dynamic_slice_in_dim(Wq, my * HD_LOCAL, HD_LOCAL, axis=1) * 0.125
    Wo_my = lax.dynamic_slice_in_dim(Wo, my * HD_LOCAL, HD_LOCAL, axis=0)
    x_b = x.astype(jnp.bfloat16)
    Wq_my = Wq_my.astype(jnp.bfloat16)
    Wo_my = Wo_my.astype(jnp.bfloat16)
    K_t = K_ext.transpose(0, 2, 1, 3).astype(jnp.bfloat16)
    V_t = V_ext.transpose(0, 2, 1, 3).astype(jnp.bfloat16)

    def body(x_ref, wq_ref, k_ref, v_ref, wo_ref, out_ref,
             q_ref, ctx_ref, mask_ref, rs1a, rs2a, rs1b, rs2b,
             send_sems, recv_sems):
        my_pos = lax.axis_index("i")
        yc = ((my_pos + 1) // 2) % 2
        xc = my_pos // 2
        p_y = my_pos ^ 1
        p_x = 3 - my_pos

        barrier_sem = pltpu.get_barrier_semaphore()
        for nbr in (p_y, p_x):
            pl.semaphore_signal(
                barrier_sem, inc=1,
                device_id=(nbr,), device_id_type=pl.DeviceIdType.MESH,
            )
        pl.semaphore_wait(barrier_sem, 2)

        qb = lax.broadcasted_iota(jnp.int32, (SQ, SKV), 0) // BLK
        kb = lax.broadcasted_iota(jnp.int32, (SQ, SKV), 1) // BLK
        keep = (qb == kb) | (kb == 0) | ((qb + kb) % 3 == 0)
        mask_ref[...] = keep.astype(jnp.float32)

        def qproj(b):
            q_ref[b] = jnp.dot(
                x_ref[b], wq_ref[...], preferred_element_type=jnp.float32
            ).astype(jnp.bfloat16)

        def chunk(b, off, rows):
            for h in range(HQ_LOCAL):
                q = q_ref[b, pl.ds(off, rows), h * DH:(h + 1) * DH]
                s = lax.dot_general(
                    q, k_ref[b, h], (((1,), (1,)), ((), ())),
                    preferred_element_type=jnp.float32,
                )
                w = jnp.exp(s) * mask_ref[pl.ds(off, rows), :]
                denom = jnp.sum(w, axis=-1, keepdims=True)
                ctx = jnp.dot(
                    w.astype(jnp.bfloat16), v_ref[b, h],
                    preferred_element_type=jnp.float32,
                ) / denom
                ctx_ref[pl.ds(off, rows), h * DH:(h + 1) * DH] = (
                    ctx.astype(jnp.bfloat16))
            out_ref[b, pl.ds(off, rows), :] = jnp.dot(
                ctx_ref[pl.ds(off, rows), :], wo_ref[...],
                preferred_element_type=jnp.float32,
            ).astype(jnp.bfloat16)

        streams = (
            (0, yc, xc, (p_y, p_x, p_x, p_y), rs1a, rs2a),
            (1, xc, yc, (p_x, p_y, p_y, p_x), rs1b, rs2b),
        )

        def make_round(s, k):
            b, h, q, parts, rs1, rs2 = streams[s]
            half0 = h * HALF
            own = half0 + q * QTR
            if k == 0:
                src = out_ref.at[b, pl.ds((1 - h) * HALF, HALF), :]
                dst = rs1.at[...]
            elif k == 1:
                src = out_ref.at[b, pl.ds(half0 + (1 - q) * QTR, QTR), :]
                dst = rs2.at[...]
            elif k == 2:
                src = out_ref.at[b, pl.ds(own, QTR), :]
                dst = out_ref.at[b, pl.ds(own, QTR), :]
            else:
                src = out_ref.at[b, pl.ds(half0, HALF), :]
                dst = out_ref.at[b, pl.ds(half0, HALF), :]
            return pltpu.make_async_remote_copy(
                src_ref=src, dst_ref=dst,
                send_sem=send_sems.at[s, k], recv_sem=recv_sems.at[s, k],
                device_id=(parts[k],), device_id_type=pl.DeviceIdType.MESH,
            )

        def apply_round(s, k):
            b, h, q, parts, rs1, rs2 = streams[s]
            half0 = h * HALF
            if k == 0:
                out_ref[b, pl.ds(half0, HALF), :] += rs1[...]
            elif k == 1:
                own = half0 + q * QTR
                out_ref[b, pl.ds(own, QTR), :] += rs2[...]

        h_a = yc
        h_b = xc

        qproj(0)
        chunk(0, (1 - h_a) * HALF, HALF)
        ra1 = make_round(0, 0)
        ra1.start()
        qproj(1)
        chunk(1, (1 - h_b) * HALF, HALF)
        rb1 = make_round(1, 0)
        rb1.start()
        chunk(0, h_a * HALF, HALF)
        ra1.wait()
        apply_round(0, 0)
        ra2 = make_round(0, 1)
        ra2.start()
        chunk(1, h_b * HALF, HALF)
        rb1.wait()
        apply_round(1, 0)
        rb2 = make_round(1, 1)
        rb2.start()
        ra2.wait()
        apply_round(0, 1)
        ra3 = make_round(0, 2)
        ra3.start()
        rb2.wait()
        apply_round(1, 1)
        rb3 = make_round(1, 2)
        rb3.start()
        ra3.wait()
        ra4 = make_round(0, 3)
        ra4.start()
        rb3.wait()
        rb4 = make_round(1, 3)
        rb4.start()
        ra4.wait()
        rb4.wait()

    return pl.pallas_call(
        body,
        out_shape=jax.ShapeDtypeStruct((B, SQ, D_MODEL), jnp.bfloat16),
        in_specs=[pl.BlockSpec(memory_space=pltpu.VMEM)] * 5,
        out_specs=pl.BlockSpec(memory_space=pltpu.VMEM),
        scratch_shapes=[
            pltpu.VMEM((B, SQ, HD_LOCAL), jnp.bfloat16),
            pltpu.VMEM((SQ, HD_LOCAL), jnp.bfloat16),
            pltpu.VMEM((SQ, SKV), jnp.float32),
            pltpu.VMEM((HALF, D_MODEL), jnp.bfloat16),
            pltpu.VMEM((QTR, D_MODEL), jnp.bfloat16),
            pltpu.VMEM((HALF, D_MODEL), jnp.bfloat16),
            pltpu.VMEM((QTR, D_MODEL), jnp.bfloat16),
            pltpu.SemaphoreType.DMA((2, 4)),
            pltpu.SemaphoreType.DMA((2, 4)),
        ],
        compiler_params=pltpu.CompilerParams(collective_id=0),
    )(x_b, Wq_my, K_t, V_t, Wo_my)
